# baseline (speedup 1.0000x reference)
import jax
import jax.numpy as jnp
from jax import lax
from jax.experimental import pallas as pl
from jax.experimental.pallas import tpu as pltpu
from jax.experimental.pallas import tpu_sc as plsc

N = 10000
IN_DIM = 128
HID = 16
OUT = 2
E = 320000

NC = 2
NS = 16
NW = NC * NS
CHUNK = 1024
CH = 10
EP = NW * CH * CHUNK
NP = 10240
RPT = NP // NS
BLK = 512


def _mesh():
    return plsc.VectorSubcoreMesh(
        core_axis_name="c", subcore_axis_name="s", num_cores=NC, num_subcores=NS
    )


DCH = EP // CHUNK // NS


def _rsqrt16(d):
    i = plsc.bitcast(d, jnp.int32)
    y = plsc.bitcast(jnp.int32(0x5F3759DF) - (i >> 1), jnp.float32)
    for _ in range(3):
        y = y * (1.5 - 0.5 * d * y * y)
    return y


def _fused_body(h1, srci, dsti, ones_h, zeros1, zeros2, a1p, degp, u,
                dz_idx, ones_v, idx_s, idx_d, rows, hbuf, ubuf, dv,
                deg_sh, a_sh, sem0, sem1):
    c = lax.axis_index("c")
    s = lax.axis_index("s")

    pltpu.sync_copy(dsti.at[pl.ds(s * DCH, DCH)], dz_idx)
    pltpu.sync_copy(ones_h, ones_v)
    pltpu.sync_copy(zeros1.at[pl.ds(s * RPT, RPT)], deg_sh.at[pl.ds(s * RPT, RPT)])
    pltpu.sync_copy(zeros2.at[pl.ds(s * RPT, RPT)], a_sh.at[pl.ds(s * RPT, RPT)])
    plsc.subcore_barrier()

    def dbody(j, carry):
        pltpu.sync_copy(ones_v, deg_sh.at[dz_idx.at[j]], add=True)
        return carry

    lax.fori_loop(0, DCH, dbody, 0)
    plsc.subcore_barrier()

    pltpu.sync_copy(deg_sh.at[pl.ds(s * RPT, RPT)], dv)
    pltpu.sync_copy(h1.at[pl.ds(s * RPT, RPT)], hbuf)

    lanes = lax.iota(jnp.int32, 16)

    def sbody(k, carry):
        y = _rsqrt16(dv[pl.ds(16 * k, 16)] + 1.0)
        rowi = 16 * k + lanes
        for j in range(HID):
            colj = jnp.full((16,), j, jnp.int32)
            vals = plsc.load_gather(hbuf, [rowi, colj]) * y
            plsc.store_scatter(ubuf, [rowi, colj], vals)
        return carry

    lax.fori_loop(0, RPT // 16, sbody, 0)
    pltpu.sync_copy(ubuf, u.at[c, pl.ds(s * RPT, RPT)])
    pltpu.sync_copy(dv, degp.at[c, pl.ds(s * RPT, RPT)])
    plsc.subcore_barrier()

    base = (c * NS + s) * CH
    pltpu.sync_copy(srci.at[pl.ds(base, CH)], idx_s)
    pltpu.sync_copy(dsti.at[pl.ds(base, CH)], idx_d)
    tab = u.at[c]
    pltpu.async_copy(tab.at[idx_s.at[0]], rows.at[0], sem0)

    def body(j, carry):
        j0 = 2 * j
        j1 = j0 + 1
        pltpu.make_async_copy(tab.at[idx_s.at[j0]], rows.at[0], sem0).wait()
        pltpu.async_copy(tab.at[idx_s.at[j1]], rows.at[1], sem1)
        pltpu.sync_copy(rows.at[0], a_sh.at[idx_d.at[j0]], add=True)
        pltpu.make_async_copy(tab.at[idx_s.at[j1]], rows.at[1], sem1).wait()

        @pl.when(j1 + 1 < CH)
        def _():
            pltpu.async_copy(tab.at[idx_s.at[j1 + 1]], rows.at[0], sem0)

        pltpu.sync_copy(rows.at[1], a_sh.at[idx_d.at[j1]], add=True)
        return carry

    lax.fori_loop(0, CH // 2, body, 0)
    plsc.subcore_barrier()
    pltpu.sync_copy(a_sh.at[pl.ds(s * RPT, RPT)], a1p.at[c, pl.ds(s * RPT, RPT)])


def _fused_pass(h1, srci, dsti, ones_h, zeros1, zeros2):
    return pl.kernel(
        _fused_body,
        out_type=[
            jax.ShapeDtypeStruct((NC, NP, HID), jnp.float32),
            jax.ShapeDtypeStruct((NC, NP), jnp.float32),
            jax.ShapeDtypeStruct((NC, NP, HID), jnp.float32),
        ],
        mesh=_mesh(),
        scratch_types=[
            pltpu.VMEM((DCH, CHUNK), jnp.int32),
            pltpu.VMEM((CHUNK,), jnp.float32),
            pltpu.VMEM((CH, CHUNK), jnp.int32),
            pltpu.VMEM((CH, CHUNK), jnp.int32),
            pltpu.VMEM((2, CHUNK, HID), jnp.float32),
            pltpu.VMEM((RPT, HID), jnp.float32),
            pltpu.VMEM((RPT, HID), jnp.float32),
            pltpu.VMEM((RPT,), jnp.float32),
            pltpu.VMEM_SHARED((NP,), jnp.float32),
            pltpu.VMEM_SHARED((NP, HID), jnp.float32),
            pltpu.SemaphoreType.DMA,
            pltpu.SemaphoreType.DMA,
        ],
        compiler_params=pltpu.CompilerParams(
            use_tc_tiling_on_sc=False, needs_layout_passes=False
        ),
    )(h1, srci, dsti, ones_h, zeros1, zeros2)


def _agg_body(g, srci, dsti, zeros_h, out, idx_s, idx_d, rows, a_sh, sem0, sem1):
    c = lax.axis_index("c")
    s = lax.axis_index("s")
    base = (c * NS + s) * CH
    pltpu.sync_copy(srci.at[pl.ds(base, CH)], idx_s)
    pltpu.sync_copy(dsti.at[pl.ds(base, CH)], idx_d)
    pltpu.sync_copy(zeros_h.at[pl.ds(s * RPT, RPT)], a_sh.at[pl.ds(s * RPT, RPT)])
    plsc.subcore_barrier()

    pltpu.async_copy(g.at[idx_s.at[0]], rows.at[0], sem0)

    def body(j, carry):
        j0 = 2 * j
        j1 = j0 + 1
        pltpu.make_async_copy(g.at[idx_s.at[j0]], rows.at[0], sem0).wait()
        pltpu.async_copy(g.at[idx_s.at[j1]], rows.at[1], sem1)
        pltpu.sync_copy(rows.at[0], a_sh.at[idx_d.at[j0]], add=True)
        pltpu.make_async_copy(g.at[idx_s.at[j1]], rows.at[1], sem1).wait()

        @pl.when(j1 + 1 < CH)
        def _():
            pltpu.async_copy(g.at[idx_s.at[j1 + 1]], rows.at[0], sem0)

        pltpu.sync_copy(rows.at[1], a_sh.at[idx_d.at[j1]], add=True)
        return carry

    lax.fori_loop(0, CH // 2, body, 0)
    plsc.subcore_barrier()
    pltpu.sync_copy(a_sh.at[pl.ds(s * RPT, RPT)], out.at[c, pl.ds(s * RPT, RPT)])


def _agg_pass(g, srci, dsti, zeros_h):
    return pl.kernel(
        _agg_body,
        out_type=jax.ShapeDtypeStruct((NC, NP, HID), jnp.float32),
        mesh=_mesh(),
        scratch_types=[
            pltpu.VMEM((CH, CHUNK), jnp.int32),
            pltpu.VMEM((CH, CHUNK), jnp.int32),
            pltpu.VMEM((2, CHUNK, HID), jnp.float32),
            pltpu.VMEM_SHARED((NP, HID), jnp.float32),
            pltpu.SemaphoreType.DMA,
            pltpu.SemaphoreType.DMA,
        ],
        compiler_params=pltpu.CompilerParams(use_tc_tiling_on_sc=False),
    )(g, srci, dsti, zeros_h)


def _dis_of(degp_blk):
    d = (degp_blk[0, :] + degp_blk[1, :]) * 0.5 + 1.0
    return lax.rsqrt(d).reshape(BLK, 1)


def _mm_body(x_ref, w1_ref, h1_ref):
    h1_ref[...] = jnp.dot(x_ref[...], w1_ref[...], preferred_element_type=jnp.float32)


def _tc_mm(xp, w1):
    grid = NP // BLK
    return pl.pallas_call(
        _mm_body,
        grid=(grid,),
        in_specs=[
            pl.BlockSpec((BLK, IN_DIM), lambda i: (i, 0)),
            pl.BlockSpec((IN_DIM, HID), lambda i: (0, 0)),
        ],
        out_specs=pl.BlockSpec((BLK, HID), lambda i: (i, 0)),
        out_shape=jax.ShapeDtypeStruct((NP, HID), jnp.float32),
    )(xp, w1)


def _tc2_body(degp_ref, a1p_ref, h1_ref, b1_ref, r1_ref, g2_ref):
    dis = _dis_of(degp_ref)
    a1 = a1p_ref[0] + a1p_ref[1]
    z = dis * a1 + (dis * dis) * h1_ref[...] + b1_ref[...]
    r = jnp.maximum(z, 0.0)
    r1_ref[...] = r
    g2_ref[...] = dis * r


def _tc2(degp, a1p, h1, b1):
    grid = NP // BLK
    return pl.pallas_call(
        _tc2_body,
        grid=(grid,),
        in_specs=[
            pl.BlockSpec((NC, BLK), lambda i: (0, i)),
            pl.BlockSpec((NC, BLK, HID), lambda i: (0, i, 0)),
            pl.BlockSpec((BLK, HID), lambda i: (i, 0)),
            pl.BlockSpec((1, HID), lambda i: (0, 0)),
        ],
        out_specs=[
            pl.BlockSpec((BLK, HID), lambda i: (i, 0)),
            pl.BlockSpec((BLK, HID), lambda i: (i, 0)),
        ],
        out_shape=[
            jax.ShapeDtypeStruct((NP, HID), jnp.float32),
            jax.ShapeDtypeStruct((NP, HID), jnp.float32),
        ],
    )(degp, a1p, h1, b1)


def _tc3_body(degp_ref, a2p_ref, r1_ref, w2_ref, b2_ref, out_ref):
    dis = _dis_of(degp_ref)
    z = dis * (a2p_ref[0] + a2p_ref[1]) + (dis * dis) * r1_ref[...]
    out_ref[...] = (
        jnp.dot(z, w2_ref[...], preferred_element_type=jnp.float32) + b2_ref[...]
    )


def _tc3(degp, a2p, r1, w2, b2):
    grid = NP // BLK
    return pl.pallas_call(
        _tc3_body,
        grid=(grid,),
        in_specs=[
            pl.BlockSpec((NC, BLK), lambda i: (0, i)),
            pl.BlockSpec((NC, BLK, HID), lambda i: (0, i, 0)),
            pl.BlockSpec((BLK, HID), lambda i: (i, 0)),
            pl.BlockSpec((HID, OUT), lambda i: (0, 0)),
            pl.BlockSpec((1, OUT), lambda i: (0, 0)),
        ],
        out_specs=pl.BlockSpec((BLK, OUT), lambda i: (i, 0)),
        out_shape=jax.ShapeDtypeStruct((NP, OUT), jnp.float32),
    )(degp, a2p, r1, w2, b2)


@jax.jit
def _run(x, edge_index, W1, b1, W2, b2):
    src = edge_index[0].astype(jnp.int32)
    dst = edge_index[1].astype(jnp.int32)
    pad = jnp.full((EP - E,), N, dtype=jnp.int32)
    srci = jnp.concatenate([src, pad]).reshape(EP // CHUNK, CHUNK)
    dsti = jnp.concatenate([dst, pad]).reshape(EP // CHUNK, CHUNK)
    xp = jnp.zeros((NP, IN_DIM), jnp.float32).at[:N].set(x)
    ones_h = jnp.ones((CHUNK,), jnp.float32)
    zeros1 = jnp.zeros((NP,), jnp.float32)
    zeros2 = jnp.zeros((NP, HID), jnp.float32)

    h1 = _tc_mm(xp, W1)
    a1p, degp, _u = _fused_pass(h1, srci, dsti, ones_h, zeros1, zeros2)
    r1, g2 = _tc2(degp, a1p, h1, b1.reshape(1, HID))
    a2p = _agg_pass(g2, srci, dsti, zeros2)
    out = _tc3(degp, a2p, r1, W2, b2.reshape(1, OUT))
    return out[:N]


def kernel(x, edge_index, W1, b1, W2, b2):
    return _run(x, edge_index, W1, b1, W2, b2)


# speedup vs baseline: 1.0027x; 1.0027x over previous
import jax
import jax.numpy as jnp
from jax import lax
from jax.experimental import pallas as pl
from jax.experimental.pallas import tpu as pltpu
from jax.experimental.pallas import tpu_sc as plsc

N = 10000
IN_DIM = 128
HID = 16
OUT = 2
E = 320000

NC = 2
NS = 16
NW = NC * NS
CHUNK = 1024
CH = 10
EP = NW * CH * CHUNK
NP = 10240
RPT = NP // NS
BLK = 512


def _mesh():
    return plsc.VectorSubcoreMesh(
        core_axis_name="c", subcore_axis_name="s", num_cores=NC, num_subcores=NS
    )


DCH = EP // CHUNK // NS


def _rsqrt16(d):
    i = plsc.bitcast(d, jnp.int32)
    y = plsc.bitcast(jnp.int32(0x5F3759DF) - (i >> 1), jnp.float32)
    for _ in range(3):
        y = y * (1.5 - 0.5 * d * y * y)
    return y


def _fused_body(h1, srci, dsti, ones_h, zeros1, zeros2, a1p, degp, u,
                dz_idx, ones_v, idx_s, idx_d, rows, hbuf, ubuf, dv,
                deg_sh, a_sh, sem0, sem1):
    c = lax.axis_index("c")
    s = lax.axis_index("s")

    pltpu.sync_copy(dsti.at[pl.ds(s * DCH, DCH)], dz_idx)
    pltpu.sync_copy(ones_h, ones_v)
    pltpu.sync_copy(zeros1.at[pl.ds(s * RPT, RPT)], deg_sh.at[pl.ds(s * RPT, RPT)])
    pltpu.sync_copy(zeros2.at[pl.ds(s * RPT, RPT)], a_sh.at[pl.ds(s * RPT, RPT)])
    pltpu.sync_copy(h1.at[pl.ds(s * RPT, RPT)], hbuf)
    base = (c * NS + s) * CH
    pltpu.sync_copy(srci.at[pl.ds(base, CH)], idx_s)
    pltpu.sync_copy(dsti.at[pl.ds(base, CH)], idx_d)
    plsc.subcore_barrier()

    def dbody(j, carry):
        pltpu.async_copy(ones_v, deg_sh.at[dz_idx.at[j]], sem0, add=True)
        return carry

    lax.fori_loop(0, DCH, dbody, 0)

    def dwait(j, carry):
        pltpu.make_async_copy(ones_v, deg_sh.at[dz_idx.at[j]], sem0).wait()
        return carry

    lax.fori_loop(0, DCH, dwait, 0)
    plsc.subcore_barrier()

    pltpu.sync_copy(deg_sh.at[pl.ds(s * RPT, RPT)], dv)

    lanes = lax.iota(jnp.int32, 16)

    def sbody(k, carry):
        y = _rsqrt16(dv[pl.ds(16 * k, 16)] + 1.0)
        rowi = 16 * k + lanes
        for j in range(HID):
            colj = jnp.full((16,), j, jnp.int32)
            vals = plsc.load_gather(hbuf, [rowi, colj]) * y
            plsc.store_scatter(ubuf, [rowi, colj], vals)
        return carry

    lax.fori_loop(0, RPT // 16, sbody, 0)
    pltpu.sync_copy(ubuf, u.at[c, pl.ds(s * RPT, RPT)])
    pltpu.sync_copy(dv, degp.at[c, pl.ds(s * RPT, RPT)])
    plsc.subcore_barrier()

    tab = u.at[c]
    pltpu.async_copy(tab.at[idx_s.at[0]], rows.at[0], sem0)

    def body(j, carry):
        j0 = 2 * j
        j1 = j0 + 1
        pltpu.make_async_copy(tab.at[idx_s.at[j0]], rows.at[0], sem0).wait()
        pltpu.async_copy(tab.at[idx_s.at[j1]], rows.at[1], sem1)
        pltpu.sync_copy(rows.at[0], a_sh.at[idx_d.at[j0]], add=True)
        pltpu.make_async_copy(tab.at[idx_s.at[j1]], rows.at[1], sem1).wait()

        @pl.when(j1 + 1 < CH)
        def _():
            pltpu.async_copy(tab.at[idx_s.at[j1 + 1]], rows.at[0], sem0)

        pltpu.sync_copy(rows.at[1], a_sh.at[idx_d.at[j1]], add=True)
        return carry

    lax.fori_loop(0, CH // 2, body, 0)
    plsc.subcore_barrier()
    pltpu.sync_copy(a_sh.at[pl.ds(s * RPT, RPT)], a1p.at[c, pl.ds(s * RPT, RPT)])


def _fused_pass(h1, srci, dsti, ones_h, zeros1, zeros2):
    return pl.kernel(
        _fused_body,
        out_type=[
            jax.ShapeDtypeStruct((NC, NP, HID), jnp.float32),
            jax.ShapeDtypeStruct((NC, NP), jnp.float32),
            jax.ShapeDtypeStruct((NC, NP, HID), jnp.float32),
        ],
        mesh=_mesh(),
        scratch_types=[
            pltpu.VMEM((DCH, CHUNK), jnp.int32),
            pltpu.VMEM((CHUNK,), jnp.float32),
            pltpu.VMEM((CH, CHUNK), jnp.int32),
            pltpu.VMEM((CH, CHUNK), jnp.int32),
            pltpu.VMEM((2, CHUNK, HID), jnp.float32),
            pltpu.VMEM((RPT, HID), jnp.float32),
            pltpu.VMEM((RPT, HID), jnp.float32),
            pltpu.VMEM((RPT,), jnp.float32),
            pltpu.VMEM_SHARED((NP,), jnp.float32),
            pltpu.VMEM_SHARED((NP, HID), jnp.float32),
            pltpu.SemaphoreType.DMA,
            pltpu.SemaphoreType.DMA,
        ],
        compiler_params=pltpu.CompilerParams(
            use_tc_tiling_on_sc=False, needs_layout_passes=False
        ),
    )(h1, srci, dsti, ones_h, zeros1, zeros2)


def _agg_body(g, srci, dsti, zeros_h, out, idx_s, idx_d, rows, a_sh, sem0, sem1):
    c = lax.axis_index("c")
    s = lax.axis_index("s")
    base = (c * NS + s) * CH
    pltpu.sync_copy(srci.at[pl.ds(base, CH)], idx_s)
    pltpu.sync_copy(dsti.at[pl.ds(base, CH)], idx_d)
    pltpu.sync_copy(zeros_h.at[pl.ds(s * RPT, RPT)], a_sh.at[pl.ds(s * RPT, RPT)])
    plsc.subcore_barrier()

    pltpu.async_copy(g.at[idx_s.at[0]], rows.at[0], sem0)

    def body(j, carry):
        j0 = 2 * j
        j1 = j0 + 1
        pltpu.make_async_copy(g.at[idx_s.at[j0]], rows.at[0], sem0).wait()
        pltpu.async_copy(g.at[idx_s.at[j1]], rows.at[1], sem1)
        pltpu.sync_copy(rows.at[0], a_sh.at[idx_d.at[j0]], add=True)
        pltpu.make_async_copy(g.at[idx_s.at[j1]], rows.at[1], sem1).wait()

        @pl.when(j1 + 1 < CH)
        def _():
            pltpu.async_copy(g.at[idx_s.at[j1 + 1]], rows.at[0], sem0)

        pltpu.sync_copy(rows.at[1], a_sh.at[idx_d.at[j1]], add=True)
        return carry

    lax.fori_loop(0, CH // 2, body, 0)
    plsc.subcore_barrier()
    pltpu.sync_copy(a_sh.at[pl.ds(s * RPT, RPT)], out.at[c, pl.ds(s * RPT, RPT)])


def _agg_pass(g, srci, dsti, zeros_h):
    return pl.kernel(
        _agg_body,
        out_type=jax.ShapeDtypeStruct((NC, NP, HID), jnp.float32),
        mesh=_mesh(),
        scratch_types=[
            pltpu.VMEM((CH, CHUNK), jnp.int32),
            pltpu.VMEM((CH, CHUNK), jnp.int32),
            pltpu.VMEM((2, CHUNK, HID), jnp.float32),
            pltpu.VMEM_SHARED((NP, HID), jnp.float32),
            pltpu.SemaphoreType.DMA,
            pltpu.SemaphoreType.DMA,
        ],
        compiler_params=pltpu.CompilerParams(use_tc_tiling_on_sc=False),
    )(g, srci, dsti, zeros_h)


def _dis_of(degp_blk):
    d = (degp_blk[0, :] + degp_blk[1, :]) * 0.5 + 1.0
    return lax.rsqrt(d).reshape(BLK, 1)


def _mm_body(x_ref, w1_ref, h1_ref):
    h1_ref[...] = jnp.dot(x_ref[...], w1_ref[...], preferred_element_type=jnp.float32)


def _tc_mm(xp, w1):
    grid = NP // BLK
    return pl.pallas_call(
        _mm_body,
        grid=(grid,),
        in_specs=[
            pl.BlockSpec((BLK, IN_DIM), lambda i: (i, 0)),
            pl.BlockSpec((IN_DIM, HID), lambda i: (0, 0)),
        ],
        out_specs=pl.BlockSpec((BLK, HID), lambda i: (i, 0)),
        out_shape=jax.ShapeDtypeStruct((NP, HID), jnp.float32),
    )(xp, w1)


def _tc2_body(degp_ref, a1p_ref, h1_ref, b1_ref, r1_ref, g2_ref):
    dis = _dis_of(degp_ref)
    a1 = a1p_ref[0] + a1p_ref[1]
    z = dis * a1 + (dis * dis) * h1_ref[...] + b1_ref[...]
    r = jnp.maximum(z, 0.0)
    r1_ref[...] = r
    g2_ref[...] = dis * r


def _tc2(degp, a1p, h1, b1):
    grid = NP // BLK
    return pl.pallas_call(
        _tc2_body,
        grid=(grid,),
        in_specs=[
            pl.BlockSpec((NC, BLK), lambda i: (0, i)),
            pl.BlockSpec((NC, BLK, HID), lambda i: (0, i, 0)),
            pl.BlockSpec((BLK, HID), lambda i: (i, 0)),
            pl.BlockSpec((1, HID), lambda i: (0, 0)),
        ],
        out_specs=[
            pl.BlockSpec((BLK, HID), lambda i: (i, 0)),
            pl.BlockSpec((BLK, HID), lambda i: (i, 0)),
        ],
        out_shape=[
            jax.ShapeDtypeStruct((NP, HID), jnp.float32),
            jax.ShapeDtypeStruct((NP, HID), jnp.float32),
        ],
    )(degp, a1p, h1, b1)


def _tc3_body(degp_ref, a2p_ref, r1_ref, w2_ref, b2_ref, out_ref):
    dis = _dis_of(degp_ref)
    z = dis * (a2p_ref[0] + a2p_ref[1]) + (dis * dis) * r1_ref[...]
    out_ref[...] = (
        jnp.dot(z, w2_ref[...], preferred_element_type=jnp.float32) + b2_ref[...]
    )


def _tc3(degp, a2p, r1, w2, b2):
    grid = NP // BLK
    return pl.pallas_call(
        _tc3_body,
        grid=(grid,),
        in_specs=[
            pl.BlockSpec((NC, BLK), lambda i: (0, i)),
            pl.BlockSpec((NC, BLK, HID), lambda i: (0, i, 0)),
            pl.BlockSpec((BLK, HID), lambda i: (i, 0)),
            pl.BlockSpec((HID, OUT), lambda i: (0, 0)),
            pl.BlockSpec((1, OUT), lambda i: (0, 0)),
        ],
        out_specs=pl.BlockSpec((BLK, OUT), lambda i: (i, 0)),
        out_shape=jax.ShapeDtypeStruct((NP, OUT), jnp.float32),
    )(degp, a2p, r1, w2, b2)


@jax.jit
def _run(x, edge_index, W1, b1, W2, b2):
    src = edge_index[0].astype(jnp.int32)
    dst = edge_index[1].astype(jnp.int32)
    pad = jnp.full((EP - E,), N, dtype=jnp.int32)
    srci = jnp.concatenate([src, pad]).reshape(EP // CHUNK, CHUNK)
    dsti = jnp.concatenate([dst, pad]).reshape(EP // CHUNK, CHUNK)
    xp = jnp.zeros((NP, IN_DIM), jnp.float32).at[:N].set(x)
    ones_h = jnp.ones((CHUNK,), jnp.float32)
    zeros1 = jnp.zeros((NP,), jnp.float32)
    zeros2 = jnp.zeros((NP, HID), jnp.float32)

    h1 = _tc_mm(xp, W1)
    a1p, degp, _u = _fused_pass(h1, srci, dsti, ones_h, zeros1, zeros2)
    r1, g2 = _tc2(degp, a1p, h1, b1.reshape(1, HID))
    a2p = _agg_pass(g2, srci, dsti, zeros2)
    out = _tc3(degp, a2p, r1, W2, b2.reshape(1, OUT))
    return out[:N]


def kernel(x, edge_index, W1, b1, W2, b2):
    return _run(x, edge_index, W1, b1, W2, b2)


# speedup vs baseline: 1.1107x; 1.1077x over previous
import jax
import jax.numpy as jnp
from jax import lax
from jax.experimental import pallas as pl
from jax.experimental.pallas import tpu as pltpu
from jax.experimental.pallas import tpu_sc as plsc

N = 10000
IN_DIM = 128
HID = 16
OUT = 2
E = 320000

NC = 2
NS = 16
NW = NC * NS
CHUNK = 640
CH = 16
NBUF = 8
EP = NW * CH * CHUNK
NP = 10240
RPT = NP // NS
BLK = 512


def _mesh():
    return plsc.VectorSubcoreMesh(
        core_axis_name="c", subcore_axis_name="s", num_cores=NC, num_subcores=NS
    )


def _deg_body(dsti, ones_h, zeros_h, out, idx_d, ones_v, deg_sh, sem):
    c = lax.axis_index("c")
    s = lax.axis_index("s")
    base = (c * NS + s) * CH
    pltpu.sync_copy(dsti.at[pl.ds(base, CH)], idx_d)
    pltpu.sync_copy(ones_h, ones_v)
    pltpu.sync_copy(zeros_h.at[pl.ds(s * RPT, RPT)], deg_sh.at[pl.ds(s * RPT, RPT)])
    plsc.subcore_barrier()

    def body(j, carry):
        pltpu.sync_copy(ones_v, deg_sh.at[idx_d.at[j]], add=True)
        return carry

    lax.fori_loop(0, CH, body, 0)
    plsc.subcore_barrier()
    pltpu.sync_copy(deg_sh.at[pl.ds(s * RPT, RPT)], out.at[c, pl.ds(s * RPT, RPT)])


def _deg_pass(dsti, ones_h, zeros_h):
    return pl.kernel(
        _deg_body,
        out_type=jax.ShapeDtypeStruct((NC, NP), jnp.float32),
        mesh=_mesh(),
        scratch_types=[
            pltpu.VMEM((CH, CHUNK), jnp.int32),
            pltpu.VMEM((CHUNK,), jnp.float32),
            pltpu.VMEM_SHARED((NP,), jnp.float32),
            pltpu.SemaphoreType.DMA,
        ],
        compiler_params=pltpu.CompilerParams(use_tc_tiling_on_sc=False),
    )(dsti, ones_h, zeros_h)


def _agg_body(g, srci, dsti, zeros_h, out, idx_s, idx_d, rows, a_sh, semg, sems):
    c = lax.axis_index("c")
    s = lax.axis_index("s")
    base = (c * NS + s) * CH
    pltpu.sync_copy(srci.at[pl.ds(base, CH)], idx_s)
    pltpu.sync_copy(dsti.at[pl.ds(base, CH)], idx_d)
    pltpu.sync_copy(zeros_h.at[pl.ds(s * RPT, RPT)], a_sh.at[pl.ds(s * RPT, RPT)])
    plsc.subcore_barrier()

    def rnd(j, carry):
        b0 = j * NBUF
        for t in range(NBUF):
            @pl.when(j > 0)
            def _():
                pltpu.make_async_copy(
                    rows.at[t], a_sh.at[idx_d.at[b0 - NBUF + t]], sems.at[t]
                ).wait()
            pltpu.async_copy(g.at[idx_s.at[b0 + t]], rows.at[t], semg.at[t])
        for t in range(NBUF):
            pltpu.make_async_copy(g.at[idx_s.at[b0 + t]], rows.at[t], semg.at[t]).wait()
            pltpu.async_copy(rows.at[t], a_sh.at[idx_d.at[b0 + t]], sems.at[t], add=True)
        return carry

    lax.fori_loop(0, CH // NBUF, rnd, 0)
    for t in range(NBUF):
        pltpu.make_async_copy(
            rows.at[t], a_sh.at[idx_d.at[CH - NBUF + t]], sems.at[t]
        ).wait()
    plsc.subcore_barrier()
    pltpu.sync_copy(a_sh.at[pl.ds(s * RPT, RPT)], out.at[c, pl.ds(s * RPT, RPT)])


def _agg_pass(g, srci, dsti, zeros_h):
    return pl.kernel(
        _agg_body,
        out_type=jax.ShapeDtypeStruct((NC, NP, HID), jnp.float32),
        mesh=_mesh(),
        scratch_types=[
            pltpu.VMEM((CH, CHUNK), jnp.int32),
            pltpu.VMEM((CH, CHUNK), jnp.int32),
            pltpu.VMEM((NBUF, CHUNK, HID), jnp.float32),
            pltpu.VMEM_SHARED((NP, HID), jnp.float32),
            pltpu.SemaphoreType.DMA((NBUF,)),
            pltpu.SemaphoreType.DMA((NBUF,)),
        ],
        compiler_params=pltpu.CompilerParams(use_tc_tiling_on_sc=False),
    )(g, srci, dsti, zeros_h)


def _dis_of(degp_blk):
    d = degp_blk[0, :] + degp_blk[1, :] + 1.0
    return lax.rsqrt(d).reshape(BLK, 1)


def _tc1_body(x_ref, w1_ref, degp_ref, h1_ref, g1_ref):
    h = jnp.dot(x_ref[...], w1_ref[...], preferred_element_type=jnp.float32)
    dis = _dis_of(degp_ref)
    h1_ref[...] = h
    g1_ref[...] = dis * h


def _tc1(xp, w1, degp):
    grid = NP // BLK
    return pl.pallas_call(
        _tc1_body,
        grid=(grid,),
        in_specs=[
            pl.BlockSpec((BLK, IN_DIM), lambda i: (i, 0)),
            pl.BlockSpec((IN_DIM, HID), lambda i: (0, 0)),
            pl.BlockSpec((NC, BLK), lambda i: (0, i)),
        ],
        out_specs=[
            pl.BlockSpec((BLK, HID), lambda i: (i, 0)),
            pl.BlockSpec((BLK, HID), lambda i: (i, 0)),
        ],
        out_shape=[
            jax.ShapeDtypeStruct((NP, HID), jnp.float32),
            jax.ShapeDtypeStruct((NP, HID), jnp.float32),
        ],
    )(xp, w1, degp)


def _tc2_body(degp_ref, a1p_ref, h1_ref, b1_ref, r1_ref, g2_ref):
    dis = _dis_of(degp_ref)
    a1 = a1p_ref[0] + a1p_ref[1]
    z = dis * a1 + (dis * dis) * h1_ref[...] + b1_ref[...]
    r = jnp.maximum(z, 0.0)
    r1_ref[...] = r
    g2_ref[...] = dis * r


def _tc2(degp, a1p, h1, b1):
    grid = NP // BLK
    return pl.pallas_call(
        _tc2_body,
        grid=(grid,),
        in_specs=[
            pl.BlockSpec((NC, BLK), lambda i: (0, i)),
            pl.BlockSpec((NC, BLK, HID), lambda i: (0, i, 0)),
            pl.BlockSpec((BLK, HID), lambda i: (i, 0)),
            pl.BlockSpec((1, HID), lambda i: (0, 0)),
        ],
        out_specs=[
            pl.BlockSpec((BLK, HID), lambda i: (i, 0)),
            pl.BlockSpec((BLK, HID), lambda i: (i, 0)),
        ],
        out_shape=[
            jax.ShapeDtypeStruct((NP, HID), jnp.float32),
            jax.ShapeDtypeStruct((NP, HID), jnp.float32),
        ],
    )(degp, a1p, h1, b1)


def _tc3_body(degp_ref, a2p_ref, r1_ref, w2_ref, b2_ref, out_ref):
    dis = _dis_of(degp_ref)
    z = dis * (a2p_ref[0] + a2p_ref[1]) + (dis * dis) * r1_ref[...]
    out_ref[...] = (
        jnp.dot(z, w2_ref[...], preferred_element_type=jnp.float32) + b2_ref[...]
    )


def _tc3(degp, a2p, r1, w2, b2):
    grid = NP // BLK
    return pl.pallas_call(
        _tc3_body,
        grid=(grid,),
        in_specs=[
            pl.BlockSpec((NC, BLK), lambda i: (0, i)),
            pl.BlockSpec((NC, BLK, HID), lambda i: (0, i, 0)),
            pl.BlockSpec((BLK, HID), lambda i: (i, 0)),
            pl.BlockSpec((HID, OUT), lambda i: (0, 0)),
            pl.BlockSpec((1, OUT), lambda i: (0, 0)),
        ],
        out_specs=pl.BlockSpec((BLK, OUT), lambda i: (i, 0)),
        out_shape=jax.ShapeDtypeStruct((NP, OUT), jnp.float32),
    )(degp, a2p, r1, w2, b2)


@jax.jit
def _run(x, edge_index, W1, b1, W2, b2):
    src = edge_index[0].astype(jnp.int32)
    dst = edge_index[1].astype(jnp.int32)
    pad = jnp.full((EP - E,), N, dtype=jnp.int32)
    srci = jnp.concatenate([src, pad]).reshape(EP // CHUNK, CHUNK)
    dsti = jnp.concatenate([dst, pad]).reshape(EP // CHUNK, CHUNK)
    xp = jnp.zeros((NP, IN_DIM), jnp.float32).at[:N].set(x)
    ones_h = jnp.ones((CHUNK,), jnp.float32)
    zeros1 = jnp.zeros((NP,), jnp.float32)
    zeros2 = jnp.zeros((NP, HID), jnp.float32)

    degp = _deg_pass(dsti, ones_h, zeros1)
    h1, g1 = _tc1(xp, W1, degp)
    a1p = _agg_pass(g1, srci, dsti, zeros2)
    r1, g2 = _tc2(degp, a1p, h1, b1.reshape(1, HID))
    a2p = _agg_pass(g2, srci, dsti, zeros2)
    out = _tc3(degp, a2p, r1, W2, b2.reshape(1, OUT))
    return out[:N]


def kernel(x, edge_index, W1, b1, W2, b2):
    return _run(x, edge_index, W1, b1, W2, b2)


# speedup vs baseline: 1.1108x; 1.0001x over previous
import jax
import jax.numpy as jnp
from jax import lax
from jax.experimental import pallas as pl
from jax.experimental.pallas import tpu as pltpu
from jax.experimental.pallas import tpu_sc as plsc

N = 10000
IN_DIM = 128
HID = 16
OUT = 2
E = 320000

NC = 2
NS = 16
NW = NC * NS
CHUNK = 640
CH = 16
NBUF = 8
EP = NW * CH * CHUNK
NP = 10240
RPT = NP // NS
BLK = 512


def _mesh():
    return plsc.VectorSubcoreMesh(
        core_axis_name="c", subcore_axis_name="s", num_cores=NC, num_subcores=NS
    )


def _deg_body(dsti, ones_h, zeros_h, out, idx_d, ones_v, deg_sh, sem):
    c = lax.axis_index("c")
    s = lax.axis_index("s")
    base = (c * NS + s) * CH
    pltpu.sync_copy(dsti.at[pl.ds(base, CH)], idx_d)
    pltpu.sync_copy(ones_h, ones_v)
    pltpu.sync_copy(zeros_h.at[pl.ds(s * RPT, RPT)], deg_sh.at[pl.ds(s * RPT, RPT)])
    plsc.subcore_barrier()

    def body(j, carry):
        pltpu.async_copy(ones_v, deg_sh.at[idx_d.at[j]], sem, add=True)
        return carry

    lax.fori_loop(0, CH, body, 0)

    def drain(j, carry):
        pltpu.make_async_copy(ones_v, deg_sh.at[idx_d.at[j]], sem).wait()
        return carry

    lax.fori_loop(0, CH, drain, 0)
    plsc.subcore_barrier()
    pltpu.sync_copy(deg_sh.at[pl.ds(s * RPT, RPT)], out.at[c, pl.ds(s * RPT, RPT)])


def _deg_pass(dsti, ones_h, zeros_h):
    return pl.kernel(
        _deg_body,
        out_type=jax.ShapeDtypeStruct((NC, NP), jnp.float32),
        mesh=_mesh(),
        scratch_types=[
            pltpu.VMEM((CH, CHUNK), jnp.int32),
            pltpu.VMEM((CHUNK,), jnp.float32),
            pltpu.VMEM_SHARED((NP,), jnp.float32),
            pltpu.SemaphoreType.DMA,
        ],
        compiler_params=pltpu.CompilerParams(use_tc_tiling_on_sc=False),
    )(dsti, ones_h, zeros_h)


def _agg_body(g, srci, dsti, zeros_h, out, idx_s, idx_d, rows, a_sh, semg, sems):
    c = lax.axis_index("c")
    s = lax.axis_index("s")
    base = (c * NS + s) * CH
    pltpu.sync_copy(srci.at[pl.ds(base, CH)], idx_s)
    pltpu.sync_copy(dsti.at[pl.ds(base, CH)], idx_d)
    pltpu.sync_copy(zeros_h.at[pl.ds(s * RPT, RPT)], a_sh.at[pl.ds(s * RPT, RPT)])
    plsc.subcore_barrier()

    def rnd(j, carry):
        b0 = j * NBUF
        for t in range(NBUF):
            @pl.when(j > 0)
            def _():
                pltpu.make_async_copy(
                    rows.at[t], a_sh.at[idx_d.at[b0 - NBUF + t]], sems.at[t]
                ).wait()
            pltpu.async_copy(g.at[idx_s.at[b0 + t]], rows.at[t], semg.at[t])
        for t in range(NBUF):
            pltpu.make_async_copy(g.at[idx_s.at[b0 + t]], rows.at[t], semg.at[t]).wait()
            pltpu.async_copy(rows.at[t], a_sh.at[idx_d.at[b0 + t]], sems.at[t], add=True)
        return carry

    lax.fori_loop(0, CH // NBUF, rnd, 0)
    for t in range(NBUF):
        pltpu.make_async_copy(
            rows.at[t], a_sh.at[idx_d.at[CH - NBUF + t]], sems.at[t]
        ).wait()
    plsc.subcore_barrier()
    pltpu.sync_copy(a_sh.at[pl.ds(s * RPT, RPT)], out.at[c, pl.ds(s * RPT, RPT)])


def _agg_pass(g, srci, dsti, zeros_h):
    return pl.kernel(
        _agg_body,
        out_type=jax.ShapeDtypeStruct((NC, NP, HID), jnp.float32),
        mesh=_mesh(),
        scratch_types=[
            pltpu.VMEM((CH, CHUNK), jnp.int32),
            pltpu.VMEM((CH, CHUNK), jnp.int32),
            pltpu.VMEM((NBUF, CHUNK, HID), jnp.float32),
            pltpu.VMEM_SHARED((NP, HID), jnp.float32),
            pltpu.SemaphoreType.DMA((NBUF,)),
            pltpu.SemaphoreType.DMA((NBUF,)),
        ],
        compiler_params=pltpu.CompilerParams(use_tc_tiling_on_sc=False),
    )(g, srci, dsti, zeros_h)


def _dis_of(degp_blk):
    d = degp_blk[0, :] + degp_blk[1, :] + 1.0
    return lax.rsqrt(d).reshape(BLK, 1)


def _tc1_body(x_ref, w1_ref, degp_ref, h1_ref, g1_ref):
    h = jnp.dot(x_ref[...], w1_ref[...], preferred_element_type=jnp.float32)
    dis = _dis_of(degp_ref)
    h1_ref[...] = h
    g1_ref[...] = dis * h


def _tc1(xp, w1, degp):
    grid = NP // BLK
    return pl.pallas_call(
        _tc1_body,
        grid=(grid,),
        in_specs=[
            pl.BlockSpec((BLK, IN_DIM), lambda i: (i, 0)),
            pl.BlockSpec((IN_DIM, HID), lambda i: (0, 0)),
            pl.BlockSpec((NC, BLK), lambda i: (0, i)),
        ],
        out_specs=[
            pl.BlockSpec((BLK, HID), lambda i: (i, 0)),
            pl.BlockSpec((BLK, HID), lambda i: (i, 0)),
        ],
        out_shape=[
            jax.ShapeDtypeStruct((NP, HID), jnp.float32),
            jax.ShapeDtypeStruct((NP, HID), jnp.float32),
        ],
    )(xp, w1, degp)


def _tc2_body(degp_ref, a1p_ref, h1_ref, b1_ref, r1_ref, g2_ref):
    dis = _dis_of(degp_ref)
    a1 = a1p_ref[0] + a1p_ref[1]
    z = dis * a1 + (dis * dis) * h1_ref[...] + b1_ref[...]
    r = jnp.maximum(z, 0.0)
    r1_ref[...] = r
    g2_ref[...] = dis * r


def _tc2(degp, a1p, h1, b1):
    grid = NP // BLK
    return pl.pallas_call(
        _tc2_body,
        grid=(grid,),
        in_specs=[
            pl.BlockSpec((NC, BLK), lambda i: (0, i)),
            pl.BlockSpec((NC, BLK, HID), lambda i: (0, i, 0)),
            pl.BlockSpec((BLK, HID), lambda i: (i, 0)),
            pl.BlockSpec((1, HID), lambda i: (0, 0)),
        ],
        out_specs=[
            pl.BlockSpec((BLK, HID), lambda i: (i, 0)),
            pl.BlockSpec((BLK, HID), lambda i: (i, 0)),
        ],
        out_shape=[
            jax.ShapeDtypeStruct((NP, HID), jnp.float32),
            jax.ShapeDtypeStruct((NP, HID), jnp.float32),
        ],
    )(degp, a1p, h1, b1)


def _tc3_body(degp_ref, a2p_ref, r1_ref, w2_ref, b2_ref, out_ref):
    dis = _dis_of(degp_ref)
    z = dis * (a2p_ref[0] + a2p_ref[1]) + (dis * dis) * r1_ref[...]
    out_ref[...] = (
        jnp.dot(z, w2_ref[...], preferred_element_type=jnp.float32) + b2_ref[...]
    )


def _tc3(degp, a2p, r1, w2, b2):
    grid = NP // BLK
    return pl.pallas_call(
        _tc3_body,
        grid=(grid,),
        in_specs=[
            pl.BlockSpec((NC, BLK), lambda i: (0, i)),
            pl.BlockSpec((NC, BLK, HID), lambda i: (0, i, 0)),
            pl.BlockSpec((BLK, HID), lambda i: (i, 0)),
            pl.BlockSpec((HID, OUT), lambda i: (0, 0)),
            pl.BlockSpec((1, OUT), lambda i: (0, 0)),
        ],
        out_specs=pl.BlockSpec((BLK, OUT), lambda i: (i, 0)),
        out_shape=jax.ShapeDtypeStruct((NP, OUT), jnp.float32),
    )(degp, a2p, r1, w2, b2)


@jax.jit
def _run(x, edge_index, W1, b1, W2, b2):
    src = edge_index[0].astype(jnp.int32)
    dst = edge_index[1].astype(jnp.int32)
    pad = jnp.full((EP - E,), N, dtype=jnp.int32)
    srci = jnp.concatenate([src, pad]).reshape(EP // CHUNK, CHUNK)
    dsti = jnp.concatenate([dst, pad]).reshape(EP // CHUNK, CHUNK)
    xp = jnp.zeros((NP, IN_DIM), jnp.float32).at[:N].set(x)
    ones_h = jnp.ones((CHUNK,), jnp.float32)
    zeros1 = jnp.zeros((NP,), jnp.float32)
    zeros2 = jnp.zeros((NP, HID), jnp.float32)

    degp = _deg_pass(dsti, ones_h, zeros1)
    h1, g1 = _tc1(xp, W1, degp)
    a1p = _agg_pass(g1, srci, dsti, zeros2)
    r1, g2 = _tc2(degp, a1p, h1, b1.reshape(1, HID))
    a2p = _agg_pass(g2, srci, dsti, zeros2)
    out = _tc3(degp, a2p, r1, W2, b2.reshape(1, OUT))
    return out[:N]


def kernel(x, edge_index, W1, b1, W2, b2):
    return _run(x, edge_index, W1, b1, W2, b2)
